# Initial kernel scaffold; baseline (speedup 1.0000x reference)
import jax
import jax.numpy as jnp
from jax.experimental import pallas as pl


def kernel(x, edge_attr, receivers, senders, params):
    raise NotImplementedError("write your pallas kernel here")


# speedup vs baseline: 1.2098x; 1.2098x over previous
import functools

import jax
import jax.numpy as jnp
from jax.experimental import pallas as pl
from jax.experimental.pallas import tpu as pltpu

LAT = 128
N_PAD = 10240
BN = 1024
BE = 2000
E_TOT = 320000


def _ln(u):
    mu = jnp.mean(u, axis=-1, keepdims=True)
    d = u - mu
    var = jnp.mean(d * d, axis=-1, keepdims=True)
    return d * jax.lax.rsqrt(var + 1e-5)


def _mm(a, b):
    return jax.lax.dot_general(a, b, (((1,), (0,)), ((), ())),
                               preferred_element_type=jnp.float32)


def _enc_node_body(x, w0, b0, w1, b1, w2, b2, w0r, w0s, xl_o, ar_o, as_o):
    h = jnp.maximum(_mm(x[...], w0[...]) + b0[...], 0.0)
    h = jnp.maximum(_mm(h, w1[...]) + b1[...], 0.0)
    xl = _ln(_mm(h, w2[...]) + b2[...])
    xl_o[...] = xl
    ar_o[...] = _mm(xl, w0r[...])
    as_o[...] = _mm(xl, w0s[...])


def _enc_edge_body(ea, w0, b0, w1, b1, w2, b2, el_o):
    h = jnp.maximum(_mm(ea[...], w0[...]) + b0[...], 0.0)
    h = jnp.maximum(_mm(h, w1[...]) + b1[...], 0.0)
    el_o[...] = _ln(_mm(h, w2[...]) + b2[...])


def _edge_body(has_prev, *refs):
    if has_prev:
        (el, ue, gr, gs, w0, b0, w1, b1, w2, b2, out) = refs
        pe = el[...] + ue[...]
    else:
        (el, gr, gs, w0, b0, w1, b1, w2, b2, out) = refs
        pe = el[...]
    h = jnp.maximum(_mm(pe, w0[...]) + gr[...] + gs[...] + b0[...], 0.0)
    h = jnp.maximum(_mm(h, w1[...]) + b1[...], 0.0)
    out[...] = _ln(_mm(h, w2[...]) + b2[...])


def _node_body(px, xl, agg, w0a, w0b, b0, w1, b1, w2, b2, w0r, w0s,
               px_o, ar_o, as_o):
    h = jnp.maximum(_mm(px[...], w0a[...]) + _mm(agg[...], w0b[...]) + b0[...],
                    0.0)
    h = jnp.maximum(_mm(h, w1[...]) + b1[...], 0.0)
    u = _ln(_mm(h, w2[...]) + b2[...])
    px_new = xl[...] + u
    px_o[...] = px_new
    ar_o[...] = _mm(px_new, w0r[...])
    as_o[...] = _mm(px_new, w0s[...])


def _dec_body(px, w0, b0, w1, b1, w2, b2, out):
    h = jnp.maximum(_mm(px[...], w0[...]) + b0[...], 0.0)
    h = jnp.maximum(_mm(h, w1[...]) + b1[...], 0.0)
    out[...] = _mm(h, w2[...]) + b2[...]


def _row_spec(bm, d):
    return pl.BlockSpec((bm, d), lambda i: (i, 0))


def _full_spec(shape):
    return pl.BlockSpec(shape, lambda i: tuple(0 for _ in shape))


def _w_specs(shapes):
    return [_full_spec(s) for s in shapes]


def _call_rows(body, n_rows, bm, row_ins, full_ins, n_out, out_d=LAT):
    grid = n_rows // bm
    in_specs = ([_row_spec(bm, a.shape[-1]) for a in row_ins]
                + _w_specs([a.shape for a in full_ins]))
    out_specs = [_row_spec(bm, out_d) for _ in range(n_out)]
    out_shape = [jax.ShapeDtypeStruct((n_rows, out_d), jnp.float32)
                 for _ in range(n_out)]
    if n_out == 1:
        out_specs, out_shape = out_specs[0], out_shape[0]
    return pl.pallas_call(
        body,
        grid=(grid,),
        in_specs=in_specs,
        out_specs=out_specs,
        out_shape=out_shape,
    )(*row_ins, *full_ins)


def _prep_mlp(p):
    return (p["w0"], p["b0"].reshape(1, -1), p["w1"], p["b1"].reshape(1, -1),
            p["w2"], p["b2"].reshape(1, -1))


def _gather(ar, as_, receivers, senders):
    return ar[receivers], as_[senders]


def _scatter(upd_e, receivers):
    agg = jax.ops.segment_sum(upd_e, receivers, num_segments=N_PAD)
    return agg


def kernel(x, edge_attr, receivers, senders, params):
    N = x.shape[0]
    x_p = jnp.pad(x, ((0, N_PAD - N), (0, 0)))

    pe0 = params["procs"][0]["edge"]
    w0r0 = pe0["w0"][LAT:2 * LAT]
    w0s0 = pe0["w0"][2 * LAT:]

    en = _prep_mlp(params["enc_node"])
    x_l, ar, as_ = _call_rows(
        functools.partial(_enc_node_body), N_PAD, BN,
        [x_p], [*en, w0r0, w0s0], 3)

    ee = _prep_mlp(params["enc_edge"])
    e_l = _call_rows(_enc_edge_body, E_TOT, BE, [edge_attr], [*ee], 1)

    pre_x = x_l
    upd_e = None
    for i, p in enumerate(params["procs"]):
        pedge = _prep_mlp(p["edge"])
        w0e = pedge[0][:LAT]
        gr, gs = _gather(ar, as_, receivers, senders)
        if upd_e is None:
            upd_e = _call_rows(
                functools.partial(_edge_body, False), E_TOT, BE,
                [e_l, gr, gs], [w0e, *pedge[1:]], 1)
        else:
            upd_e = _call_rows(
                functools.partial(_edge_body, True), E_TOT, BE,
                [e_l, upd_e, gr, gs], [w0e, *pedge[1:]], 1)
        agg = _scatter(upd_e, receivers)

        pnode = _prep_mlp(p["node"])
        w0a = pnode[0][:LAT]
        w0b = pnode[0][LAT:]
        if i + 1 < len(params["procs"]):
            pe_next = params["procs"][i + 1]["edge"]["w0"]
            w0rn = pe_next[LAT:2 * LAT]
            w0sn = pe_next[2 * LAT:]
        else:
            w0rn = w0r0
            w0sn = w0s0
        pre_x, ar, as_ = _call_rows(
            _node_body, N_PAD, BN,
            [pre_x, x_l, agg], [w0a, w0b, *pnode[1:], w0rn, w0sn], 3)

    dec = _prep_mlp(params["dec"])
    w2d = jnp.pad(dec[4], ((0, 0), (0, LAT - dec[4].shape[1])))
    b2d = jnp.pad(dec[5], ((0, 0), (0, LAT - dec[5].shape[1])))
    out = _call_rows(_dec_body, N_PAD, BN,
                     [pre_x], [dec[0], dec[1], dec[2], dec[3], w2d, b2d], 1)
    return out[:N, :3]


# speedup vs baseline: 3.0674x; 2.5354x over previous
import functools

import jax
import jax.numpy as jnp
from jax import lax
from jax.experimental import pallas as pl
from jax.experimental.pallas import tpu as pltpu
from jax.experimental.pallas import tpu_sc as plsc

LAT = 128
N_PAD = 10240
BN = 1024
BE = 2000
E_TOT = 320000


def _ln(u):
    mu = jnp.mean(u, axis=-1, keepdims=True)
    d = u - mu
    var = jnp.mean(d * d, axis=-1, keepdims=True)
    return d * jax.lax.rsqrt(var + 1e-5)


def _mm(a, b):
    return jax.lax.dot_general(a, b, (((1,), (0,)), ((), ())),
                               preferred_element_type=jnp.float32)


def _enc_node_body(x, w0, b0, w1, b1, w2, b2, w0r, w0s, xl_o, ar_o, as_o):
    h = jnp.maximum(_mm(x[...], w0[...]) + b0[...], 0.0)
    h = jnp.maximum(_mm(h, w1[...]) + b1[...], 0.0)
    xl = _ln(_mm(h, w2[...]) + b2[...])
    xl_o[...] = xl
    ar_o[...] = _mm(xl, w0r[...])
    as_o[...] = _mm(xl, w0s[...])


def _enc_edge_body(ea, w0, b0, w1, b1, w2, b2, el_o):
    h = jnp.maximum(_mm(ea[...], w0[...]) + b0[...], 0.0)
    h = jnp.maximum(_mm(h, w1[...]) + b1[...], 0.0)
    el_o[...] = _ln(_mm(h, w2[...]) + b2[...])


def _edge_body(has_prev, *refs):
    if has_prev:
        (el, ue, gr, gs, w0, b0, w1, b1, w2, b2, out) = refs
        pe = el[...] + ue[...]
    else:
        (el, gr, gs, w0, b0, w1, b1, w2, b2, out) = refs
        pe = el[...]
    h = jnp.maximum(_mm(pe, w0[...]) + gr[...] + gs[...] + b0[...], 0.0)
    h = jnp.maximum(_mm(h, w1[...]) + b1[...], 0.0)
    out[...] = _ln(_mm(h, w2[...]) + b2[...])


def _node_body(px, xl, agg0, agg1, w0a, w0b, b0, w1, b1, w2, b2, w0r, w0s,
               px_o, ar_o, as_o):
    agg = agg0[...] + agg1[...]
    h = jnp.maximum(_mm(px[...], w0a[...]) + _mm(agg, w0b[...]) + b0[...],
                    0.0)
    h = jnp.maximum(_mm(h, w1[...]) + b1[...], 0.0)
    u = _ln(_mm(h, w2[...]) + b2[...])
    px_new = xl[...] + u
    px_o[...] = px_new
    ar_o[...] = _mm(px_new, w0r[...])
    as_o[...] = _mm(px_new, w0s[...])


def _dec_body(px, w0, b0, w1, b1, w2, b2, out):
    h = jnp.maximum(_mm(px[...], w0[...]) + b0[...], 0.0)
    h = jnp.maximum(_mm(h, w1[...]) + b1[...], 0.0)
    out[...] = _mm(h, w2[...]) + b2[...]


def _row_spec(bm, d):
    return pl.BlockSpec((bm, d), lambda i: (i, 0))


def _full_spec(shape):
    return pl.BlockSpec(shape, lambda i: tuple(0 for _ in shape))


def _w_specs(shapes):
    return [_full_spec(s) for s in shapes]


def _call_rows(body, n_rows, bm, row_ins, full_ins, n_out, out_d=LAT):
    grid = n_rows // bm
    in_specs = ([_row_spec(bm, a.shape[-1]) for a in row_ins]
                + _w_specs([a.shape for a in full_ins]))
    out_specs = [_row_spec(bm, out_d) for _ in range(n_out)]
    out_shape = [jax.ShapeDtypeStruct((n_rows, out_d), jnp.float32)
                 for _ in range(n_out)]
    if n_out == 1:
        out_specs, out_shape = out_specs[0], out_shape[0]
    return pl.pallas_call(
        body,
        grid=(grid,),
        in_specs=in_specs,
        out_specs=out_specs,
        out_shape=out_shape,
    )(*row_ins, *full_ins)


def _prep_mlp(p):
    return (p["w0"], p["b0"].reshape(1, -1), p["w1"], p["b1"].reshape(1, -1),
            p["w2"], p["b2"].reshape(1, -1))


_NC, _NS = 2, 16
_NW = _NC * _NS
_C = 128
_NCHUNK = E_TOT // _C
_JMAX = -(-_NCHUNK // _NW)
_RPT = N_PAD // _NS


def _sc_mesh():
    return plsc.VectorSubcoreMesh(core_axis_name="c", subcore_axis_name="s")


def _gather(ar, as_, rcv2, snd2):
    @functools.partial(
        pl.kernel,
        out_type=[jax.ShapeDtypeStruct((E_TOT, LAT), jnp.float32),
                  jax.ShapeDtypeStruct((E_TOT, LAT), jnp.float32)],
        mesh=_sc_mesh(),
        scratch_types=[
            pltpu.VMEM((_C,), jnp.int32),
            pltpu.VMEM((_C,), jnp.int32),
            pltpu.VMEM((_C, LAT), jnp.float32),
            pltpu.VMEM((_C, LAT), jnp.float32),
            pltpu.SemaphoreType.DMA,
            pltpu.SemaphoreType.DMA,
        ],
    )
    def k(ar_h, as_h, rcv_h, snd_h, gr_h, gs_h, ir, is_, rr, rs, sr, ss):
        w = lax.axis_index("s") * _NC + lax.axis_index("c")

        def body(j, carry):
            c = w + _NW * j

            @pl.when(c < _NCHUNK)
            def _():
                pltpu.sync_copy(rcv_h.at[c], ir)
                pltpu.sync_copy(snd_h.at[c], is_)
                d1 = pltpu.async_copy(ar_h.at[ir], rr, sr)
                d2 = pltpu.async_copy(as_h.at[is_], rs, ss)
                d1.wait()
                d2.wait()
                base = c * _C
                pltpu.sync_copy(rr, gr_h.at[pl.ds(base, _C)])
                pltpu.sync_copy(rs, gs_h.at[pl.ds(base, _C)])
            return carry

        lax.fori_loop(0, _JMAX, body, 0)

    return k(ar, as_, rcv2, snd2)


def _scatter(upd_e, rcv2):
    @functools.partial(
        pl.kernel,
        out_type=[jax.ShapeDtypeStruct((N_PAD, LAT), jnp.float32),
                  jax.ShapeDtypeStruct((N_PAD, LAT), jnp.float32)],
        mesh=_sc_mesh(),
        scratch_types=[
            pltpu.VMEM((_C,), jnp.int32),
            pltpu.VMEM((_C, LAT), jnp.float32),
            pltpu.VMEM_SHARED((N_PAD, LAT), jnp.float32),
        ],
    )
    def k(ue_h, rcv_h, out0_h, out1_h, idx_v, rows_v, agg_sh):
        cidx = lax.axis_index("c")
        sidx = lax.axis_index("s")
        w = sidx * _NC + cidx
        zero16 = jnp.zeros((16,), jnp.float32)

        def zbody(i, carry):
            for jj in range(LAT // 16):
                rows_v[i, pl.ds(jj * 16, 16)] = zero16
            return carry

        lax.fori_loop(0, _C, zbody, 0)
        for t in range(_RPT // _C):
            pltpu.sync_copy(rows_v, agg_sh.at[pl.ds(sidx * _RPT + t * _C, _C)])
        plsc.subcore_barrier()

        def body(j, carry):
            c = w + _NW * j

            @pl.when(c < _NCHUNK)
            def _():
                pltpu.sync_copy(rcv_h.at[c], idx_v)
                pltpu.sync_copy(ue_h.at[pl.ds(c * _C, _C)], rows_v)
                pltpu.sync_copy(rows_v, agg_sh.at[idx_v], add=True)
            return carry

        lax.fori_loop(0, _JMAX, body, 0)
        plsc.subcore_barrier()

        for t in range(_RPT // _C):
            lo = sidx * _RPT + t * _C
            pltpu.sync_copy(agg_sh.at[pl.ds(lo, _C)], rows_v)

            @pl.when(cidx == 0)
            def _():
                pltpu.sync_copy(rows_v, out0_h.at[pl.ds(lo, _C)])

            @pl.when(cidx == 1)
            def _():
                pltpu.sync_copy(rows_v, out1_h.at[pl.ds(lo, _C)])

    return k(upd_e, rcv2)


def kernel(x, edge_attr, receivers, senders, params):
    N = x.shape[0]
    x_p = jnp.pad(x, ((0, N_PAD - N), (0, 0)))
    rcv2 = receivers.astype(jnp.int32).reshape(_NCHUNK, _C)
    snd2 = senders.astype(jnp.int32).reshape(_NCHUNK, _C)

    pe0 = params["procs"][0]["edge"]
    w0r0 = pe0["w0"][LAT:2 * LAT]
    w0s0 = pe0["w0"][2 * LAT:]

    en = _prep_mlp(params["enc_node"])
    x_l, ar, as_ = _call_rows(
        functools.partial(_enc_node_body), N_PAD, BN,
        [x_p], [*en, w0r0, w0s0], 3)

    ee = _prep_mlp(params["enc_edge"])
    e_l = _call_rows(_enc_edge_body, E_TOT, BE, [edge_attr], [*ee], 1)

    pre_x = x_l
    upd_e = None
    for i, p in enumerate(params["procs"]):
        pedge = _prep_mlp(p["edge"])
        w0e = pedge[0][:LAT]
        gr, gs = _gather(ar, as_, rcv2, snd2)
        if upd_e is None:
            upd_e = _call_rows(
                functools.partial(_edge_body, False), E_TOT, BE,
                [e_l, gr, gs], [w0e, *pedge[1:]], 1)
        else:
            upd_e = _call_rows(
                functools.partial(_edge_body, True), E_TOT, BE,
                [e_l, upd_e, gr, gs], [w0e, *pedge[1:]], 1)
        agg0, agg1 = _scatter(upd_e, rcv2)

        pnode = _prep_mlp(p["node"])
        w0a = pnode[0][:LAT]
        w0b = pnode[0][LAT:]
        if i + 1 < len(params["procs"]):
            pe_next = params["procs"][i + 1]["edge"]["w0"]
            w0rn = pe_next[LAT:2 * LAT]
            w0sn = pe_next[2 * LAT:]
        else:
            w0rn = w0r0
            w0sn = w0s0
        pre_x, ar, as_ = _call_rows(
            _node_body, N_PAD, BN,
            [pre_x, x_l, agg0, agg1], [w0a, w0b, *pnode[1:], w0rn, w0sn], 3)

    dec = _prep_mlp(params["dec"])
    w2d = jnp.pad(dec[4], ((0, 0), (0, LAT - dec[4].shape[1])))
    b2d = jnp.pad(dec[5], ((0, 0), (0, LAT - dec[5].shape[1])))
    out = _call_rows(_dec_body, N_PAD, BN,
                     [pre_x], [dec[0], dec[1], dec[2], dec[3], w2d, b2d], 1)
    return out[:N, :3]
